# baseline (speedup 1.0000x reference)
import functools

import jax
import jax.numpy as jnp
import numpy as np
from jax import lax
from jax.experimental import pallas as pl
from jax.experimental.pallas import tpu as pltpu
from jax.experimental.pallas import tpu_sc as plsc

NC = 2
NS = 16
NW = NC * NS
CH = 80
CHG = 128

NSLAB = 1

NG = 16
_OFFS = np.linspace(0.0, 100.0, NG).astype(np.float32)
_COEFF = np.float32(-0.5 / (_OFFS[1] - _OFFS[0]) ** 2)


def _gather_body(p_hbm, q_hbm, x0_hbm, x1_hbm, x2_hbm, dst_hbm, src_hbm,
                 g_hbm, d2_hbm,
                 idxd, idxs, bufp0, bufq0, bufp1, bufq1, d2v0, d2v1,
                 x0v, x1v, x2v, semg0, semg1, semw0, semw1):
    E = dst_hbm.shape[0]
    epw = E // NW
    nch = epw // CHG
    c = lax.axis_index("c")
    s = lax.axis_index("s")
    wid = s * NC + c
    base0 = wid * epw

    bufp = (bufp0, bufp1)
    bufq = (bufq0, bufq1)
    d2v = (d2v0, d2v1)
    semg = (semg0, semg1)
    semw = (semw0, semw1)

    pltpu.sync_copy(x0_hbm, x0v)
    pltpu.sync_copy(x1_hbm, x1v)
    pltpu.sync_copy(x2_hbm, x2v)
    pltpu.sync_copy(dst_hbm.at[pl.ds(base0, epw)], idxd)
    pltpu.sync_copy(src_hbm.at[pl.ds(base0, epw)], idxs)

    def gather_descs(k, b):
        off = k * CHG
        return (pltpu.make_async_copy(
                    p_hbm.at[idxd.at[pl.ds(off, CHG)]], bufp[b], semg[b]),
                pltpu.make_async_copy(
                    q_hbm.at[idxs.at[pl.ds(off, CHG)]], bufq[b], semg[b]))

    def wb_descs(k, b):
        base = base0 + k * CHG
        return (pltpu.make_async_copy(bufp[b], g_hbm.at[pl.ds(base, CHG)],
                                      semw[b]),
                pltpu.make_async_copy(d2v[b], d2_hbm.at[pl.ds(base, CHG)],
                                      semw[b]))

    def process(k, b):
        for dsc in gather_descs(k, b):
            dsc.wait()
        bp, bq, dv = bufp[b], bufq[b], d2v[b]

        def row(r, _):
            for j in range(8):
                sl = pl.ds(j * 16, 16)
                bp[r, sl] = bp[r, sl] + bq[r, sl]
            return 0

        lax.fori_loop(0, CHG, row, 0, unroll=2)

        def grp(g, _):
            sl = pl.ds(k * CHG + g * 16, 16)
            dvx = idxd[sl]
            svx = idxs[sl]
            d0 = plsc.load_gather(x0v, [dvx]) - plsc.load_gather(x0v, [svx])
            d1 = plsc.load_gather(x1v, [dvx]) - plsc.load_gather(x1v, [svx])
            d2 = plsc.load_gather(x2v, [dvx]) - plsc.load_gather(x2v, [svx])
            dv[pl.ds(g * 16, 16)] = d0 * d0 + d1 * d1 + d2 * d2
            return 0

        lax.fori_loop(0, CHG // 16, grp, 0, unroll=4)
        for dsc in wb_descs(k, b):
            dsc.start()

    npair = nch // 2
    assert nch == 2 * npair
    for dsc in gather_descs(0, 0):
        dsc.start()

    def pair(p, _):
        k = 2 * p

        @pl.when(p > 0)
        def _():
            for dsc in wb_descs(k - 1, 1):
                dsc.wait()

        for dsc in gather_descs(k + 1, 1):
            dsc.start()
        process(k, 0)
        for dsc in wb_descs(k, 0):
            dsc.wait()

        @pl.when(p + 1 < npair)
        def _():
            for dsc in gather_descs(k + 2, 0):
                dsc.start()

        process(k + 1, 1)
        return 0

    lax.fori_loop(0, npair, pair, 0)
    for dsc in wb_descs(nch - 1, 1):
        dsc.wait()


def _scatter_body(*refs):
    s_hbms = refs[:NSLAB]
    dst_hbms = refs[NSLAB:2 * NSLAB]
    out_hbm = refs[2 * NSLAB]
    sbuf0, idxb0, sbuf1, idxb1, zb, acc, seml0, seml1 = refs[2 * NSLAB + 1:]
    sbuf = (sbuf0, sbuf1)
    idxb = (idxb0, idxb1)
    seml = (seml0, seml1)
    N = acc.shape[0]
    rpt = 8 * (N // (8 * NS))
    rem = N - rpt * NS
    c = lax.axis_index("c")
    s = lax.axis_index("s")
    base_r = s * rpt

    def zrow(r, _):
        for k in range(8):
            zb[r, pl.ds(k * 16, 16)] = jnp.zeros((16,), jnp.float32)
        return 0

    lax.fori_loop(0, zb.shape[0], zrow, 0)

    def zcopy(j, _):
        pltpu.sync_copy(zb, acc.at[pl.ds(base_r + j * zb.shape[0],
                                         zb.shape[0])])
        return 0

    lax.fori_loop(0, rpt // zb.shape[0], zcopy, 0)

    @pl.when(s == NS - 1)
    def _():
        pltpu.sync_copy(zb, acc.at[pl.ds(N - rem, rem)])

    plsc.subcore_barrier()

    for s_hbm, dst_hbm in zip(s_hbms, dst_hbms):
        es = dst_hbm.shape[0]
        epc = es // NC
        ept = epc // NS
        nch = ept // CH
        base0 = c * epc + s * ept

        def load_descs(k, b, s_hbm=s_hbm, dst_hbm=dst_hbm, base0=base0):
            base = base0 + k * CH
            return (pltpu.make_async_copy(s_hbm.at[pl.ds(base, CH)], sbuf[b],
                                          seml[b]),
                    pltpu.make_async_copy(dst_hbm.at[pl.ds(base, CH)],
                                          idxb[b], seml[b]))

        def step(k, b, load_descs=load_descs, nch=nch):
            for dsc in load_descs(k, b):
                dsc.wait()
            pltpu.sync_copy(sbuf[b], acc.at[idxb[b]], add=True)

            @pl.when(k + 2 < nch)
            def _():
                for dsc in load_descs(k + 2, b):
                    dsc.start()

        for b in range(2):
            for dsc in load_descs(b, b):
                dsc.start()

        def pair(p, _, step=step):
            step(2 * p, 0)
            step(2 * p + 1, 1)
            return 0

        lax.fori_loop(0, nch // 2, pair, 0)
        if nch % 2:
            step(nch - 1, 0)
    plsc.subcore_barrier()
    pltpu.sync_copy(acc.at[pl.ds(base_r, rpt)],
                    out_hbm.at[c, pl.ds(base_r, rpt)])

    @pl.when(s == NS - 1)
    def _():
        pltpu.sync_copy(acc.at[pl.ds(N - rem, rem)],
                        out_hbm.at[c, pl.ds(N - rem, rem)])


def _pq_body(h_ref, whi_ref, whj_ref, p_ref, q_ref):
    h = h_ref[...]
    p_ref[...] = jnp.dot(h, whi_ref[...], preferred_element_type=jnp.float32)
    q_ref[...] = jnp.dot(h, whj_ref[...], preferred_element_type=jnp.float32)


def _edge_body(g_ref, d2_ref, ea_ref, wa_ref, wr_ref, we2_ref, winf_ref,
               be1_ref, be2_ref, binf_ref, offs_ref, s_ref):
    nrow = d2_ref.shape[0]
    parts = []
    for j in range(nrow):
        d2j = d2_ref[j]
        dj = jnp.sqrt(jnp.maximum(d2j, 1e-12))
        rftj = jnp.exp(_COEFF * (dj - offs_ref[...]) ** 2)
        parts.append(lax.dot_general(
            rftj, wr_ref[...], (((0,), (0,)), ((), ())),
            preferred_element_type=jnp.float32))
    rfc = jnp.concatenate(parts, axis=0)
    pre = (jnp.dot(ea_ref[...], wa_ref[...], preferred_element_type=jnp.float32)
           + rfc + g_ref[...] + be1_ref[...])
    h1 = jnp.maximum(pre, 0.0)
    mij = jnp.maximum(
        jnp.dot(h1, we2_ref[...], preferred_element_type=jnp.float32)
        + be2_ref[...], 0.0)
    logit = (jnp.dot(mij, winf_ref[...], preferred_element_type=jnp.float32)
             + binf_ref[...])
    s_ref[...] = mij * jax.nn.sigmoid(logit)


def _node_body(m_ref, h_ref, wna_ref, wnb_ref, wn2_ref,
               bn1_ref, bn2_ref, o_ref):
    mi = m_ref[0] + m_ref[1]
    a = jnp.maximum(
        jnp.dot(mi, wna_ref[...], preferred_element_type=jnp.float32)
        + jnp.dot(h_ref[...], wnb_ref[...], preferred_element_type=jnp.float32)
        + bn1_ref[...], 0.0)
    o_ref[...] = (jnp.dot(a, wn2_ref[...], preferred_element_type=jnp.float32)
                  + bn2_ref[...])


def _full(shape):
    return pl.BlockSpec(shape, lambda i: (0,) * len(shape))


def kernel(h, x, edge_index, edge_attr, W_e1, b_e1, W_e2, b_e2, W_inf, b_inf,
           W_n1, b_n1, W_n2, b_n2):
    N, H = h.shape
    E = edge_index.shape[1]
    EF = edge_attr.shape[1]
    dst = edge_index[0]
    src = edge_index[1]

    W_a = W_e1[:EF]
    W_r = W_e1[EF:EF + NG]
    W_hi = W_e1[EF + NG:EF + NG + H]
    W_hj = W_e1[EF + NG + H:]

    f32 = jnp.float32
    BN = 1000

    P, Q = pl.pallas_call(
        _pq_body,
        grid=(N // BN,),
        in_specs=[pl.BlockSpec((BN, H), lambda i: (i, 0)),
                  _full((H, H)), _full((H, H))],
        out_specs=[pl.BlockSpec((BN, H), lambda i: (i, 0)),
                   pl.BlockSpec((BN, H), lambda i: (i, 0))],
        out_shape=[jax.ShapeDtypeStruct((N, H), f32),
                   jax.ShapeDtypeStruct((N, H), f32)],
    )(h, W_hi, W_hj)

    x0 = x[:, 0]
    x1 = x[:, 1]
    x2 = x[:, 2]

    mesh = plsc.VectorSubcoreMesh(core_axis_name="c", subcore_axis_name="s",
                                  num_cores=NC, num_subcores=NS)
    sc_params = pltpu.CompilerParams(needs_layout_passes=False)
    BE = 2560
    epw_pad = -(-(E // NW) // (2 * CHG)) * (2 * CHG)
    E_pad = NW * epw_pad
    zpad = jnp.zeros((E_pad - E,), jnp.int32)
    dstp = jnp.concatenate([dst, zpad])
    srcp = jnp.concatenate([src, zpad])

    G, d2 = pl.kernel(
        _gather_body,
        compiler_params=sc_params,
        out_type=(jax.ShapeDtypeStruct((E_pad, H), f32),
                  jax.ShapeDtypeStruct((E_pad,), f32)),
        mesh=mesh,
        scratch_types=[
            pltpu.VMEM((epw_pad,), jnp.int32),
            pltpu.VMEM((epw_pad,), jnp.int32),
            pltpu.VMEM((CHG, H), f32),
            pltpu.VMEM((CHG, H), f32),
            pltpu.VMEM((CHG, H), f32),
            pltpu.VMEM((CHG, H), f32),
            pltpu.VMEM((CHG,), f32),
            pltpu.VMEM((CHG,), f32),
            pltpu.VMEM((N,), f32),
            pltpu.VMEM((N,), f32),
            pltpu.VMEM((N,), f32),
            pltpu.SemaphoreType.DMA,
            pltpu.SemaphoreType.DMA,
            pltpu.SemaphoreType.DMA,
            pltpu.SemaphoreType.DMA,
        ],
    )(P, Q, x0, x1, x2, dstp, srcp)

    d2r = jnp.reshape(d2, (E_pad // 128, 1, 128))
    s = pl.pallas_call(
        _edge_body,
        grid=(E // BE,),
        in_specs=[pl.BlockSpec((BE, H), lambda i: (i, 0)),
                  pl.BlockSpec((BE // 128, 1, 128), lambda i: (i, 0, 0)),
                  pl.BlockSpec((BE, EF), lambda i: (i, 0)),
                  _full((EF, H)), _full((NG, H)), _full((H, H)),
                  _full((H, 128)), _full((1, H)), _full((1, H)),
                  _full((1, 128)), _full((NG, 128))],
        out_specs=pl.BlockSpec((BE, H), lambda i: (i, 0)),
        out_shape=jax.ShapeDtypeStruct((E, H), f32),
    )(G, d2r, edge_attr, W_a, W_r, W_e2,
      jnp.broadcast_to(W_inf, (H, 128)), b_e1.reshape(1, H),
      b_e2.reshape(1, H), jnp.broadcast_to(b_inf.reshape(1, 1), (1, 128)),
      jnp.broadcast_to(jnp.asarray(_OFFS).reshape(NG, 1), (NG, 128)))

    mi2 = pl.kernel(
        _scatter_body,
        compiler_params=sc_params,
        out_type=jax.ShapeDtypeStruct((NC, N, H), f32),
        mesh=mesh,
        scratch_types=[
            pltpu.VMEM((CH, H), f32),
            pltpu.VMEM((CH,), jnp.int32),
            pltpu.VMEM((CH, H), f32),
            pltpu.VMEM((CH,), jnp.int32),
            pltpu.VMEM((16, H), f32),
            pltpu.VMEM_SHARED((N, H), f32),
            pltpu.SemaphoreType.DMA,
            pltpu.SemaphoreType.DMA,
        ],
    )(s, dst)

    out = pl.pallas_call(
        _node_body,
        grid=(N // BN,),
        in_specs=[pl.BlockSpec((NC, BN, H), lambda i: (0, i, 0)),
                  pl.BlockSpec((BN, H), lambda i: (i, 0)),
                  _full((H, H)), _full((H, H)), _full((H, H)),
                  _full((1, H)), _full((1, H))],
        out_specs=pl.BlockSpec((BN, H), lambda i: (i, 0)),
        out_shape=jax.ShapeDtypeStruct((N, H), f32),
    )(mi2, h, W_n1[:H], W_n1[H:], W_n2,
      b_n1.reshape(1, H), b_n2.reshape(1, H))

    return (out, x)


# speedup vs baseline: 1.1580x; 1.1580x over previous
import functools

import jax
import jax.numpy as jnp
import numpy as np
from jax import lax
from jax.experimental import pallas as pl
from jax.experimental.pallas import tpu as pltpu
from jax.experimental.pallas import tpu_sc as plsc

NC = 2
NS = 16
NW = NC * NS
CH = 80
CHG = 64

NSLAB = 1

NG = 16
_OFFS = np.linspace(0.0, 100.0, NG).astype(np.float32)
_COEFF = np.float32(-0.5 / (_OFFS[1] - _OFFS[0]) ** 2)


def _gather_body(p_hbm, q_hbm, x0_hbm, x1_hbm, x2_hbm, dst_hbm, src_hbm,
                 g_hbm, d2_hbm,
                 idxd, idxs, bufp0, bufq0, bufp1, bufq1, d2v0, d2v1,
                 x0v, x1v, x2v, semg0, semg1, semw0, semw1):
    E = dst_hbm.shape[0]
    epw = E // NW
    nch = epw // CHG
    c = lax.axis_index("c")
    s = lax.axis_index("s")
    wid = s * NC + c
    base0 = wid * epw

    bufp = (bufp0, bufp1)
    bufq = (bufq0, bufq1)
    d2v = (d2v0, d2v1)
    semg = (semg0, semg1)
    semw = (semw0, semw1)

    pltpu.sync_copy(x0_hbm, x0v)
    pltpu.sync_copy(x1_hbm, x1v)
    pltpu.sync_copy(x2_hbm, x2v)
    pltpu.sync_copy(dst_hbm.at[pl.ds(base0, epw)], idxd)
    pltpu.sync_copy(src_hbm.at[pl.ds(base0, epw)], idxs)

    def gather_descs(k, b):
        off = k * CHG
        return (pltpu.make_async_copy(
                    p_hbm.at[idxd.at[pl.ds(off, CHG)]], bufp[b], semg[b]),
                pltpu.make_async_copy(
                    q_hbm.at[idxs.at[pl.ds(off, CHG)]], bufq[b], semg[b]))

    def wb_descs(k, b):
        base = base0 + k * CHG
        return (pltpu.make_async_copy(bufp[b], g_hbm.at[pl.ds(base, CHG)],
                                      semw[b]),
                pltpu.make_async_copy(d2v[b], d2_hbm.at[pl.ds(base, CHG)],
                                      semw[b]))

    def process(k, b):
        for dsc in gather_descs(k, b):
            dsc.wait()
        bp, bq, dv = bufp[b], bufq[b], d2v[b]

        def row(r, _):
            for j in range(8):
                sl = pl.ds(j * 16, 16)
                bp[r, sl] = bp[r, sl] + bq[r, sl]
            return 0

        lax.fori_loop(0, CHG, row, 0, unroll=2)

        def grp(g, _):
            sl = pl.ds(k * CHG + g * 16, 16)
            dvx = idxd[sl]
            svx = idxs[sl]
            d0 = plsc.load_gather(x0v, [dvx]) - plsc.load_gather(x0v, [svx])
            d1 = plsc.load_gather(x1v, [dvx]) - plsc.load_gather(x1v, [svx])
            d2 = plsc.load_gather(x2v, [dvx]) - plsc.load_gather(x2v, [svx])
            dv[pl.ds(g * 16, 16)] = d0 * d0 + d1 * d1 + d2 * d2
            return 0

        lax.fori_loop(0, CHG // 16, grp, 0, unroll=4)
        for dsc in wb_descs(k, b):
            dsc.start()

    npair = nch // 2
    assert nch == 2 * npair
    for dsc in gather_descs(0, 0):
        dsc.start()

    def pair(p, _):
        k = 2 * p

        @pl.when(p > 0)
        def _():
            for dsc in wb_descs(k - 1, 1):
                dsc.wait()

        for dsc in gather_descs(k + 1, 1):
            dsc.start()
        process(k, 0)
        for dsc in wb_descs(k, 0):
            dsc.wait()

        @pl.when(p + 1 < npair)
        def _():
            for dsc in gather_descs(k + 2, 0):
                dsc.start()

        process(k + 1, 1)
        return 0

    lax.fori_loop(0, npair, pair, 0)
    for dsc in wb_descs(nch - 1, 1):
        dsc.wait()


def _scatter_body(*refs):
    s_hbms = refs[:NSLAB]
    dst_hbms = refs[NSLAB:2 * NSLAB]
    out_hbm = refs[2 * NSLAB]
    sbuf0, idxb0, sbuf1, idxb1, zb, acc, seml0, seml1 = refs[2 * NSLAB + 1:]
    sbuf = (sbuf0, sbuf1)
    idxb = (idxb0, idxb1)
    seml = (seml0, seml1)
    N = acc.shape[0]
    rpt = 8 * (N // (8 * NS))
    rem = N - rpt * NS
    c = lax.axis_index("c")
    s = lax.axis_index("s")
    base_r = s * rpt

    def zrow(r, _):
        for k in range(8):
            zb[r, pl.ds(k * 16, 16)] = jnp.zeros((16,), jnp.float32)
        return 0

    lax.fori_loop(0, zb.shape[0], zrow, 0)

    def zcopy(j, _):
        pltpu.sync_copy(zb, acc.at[pl.ds(base_r + j * zb.shape[0],
                                         zb.shape[0])])
        return 0

    lax.fori_loop(0, rpt // zb.shape[0], zcopy, 0)

    @pl.when(s == NS - 1)
    def _():
        pltpu.sync_copy(zb, acc.at[pl.ds(N - rem, rem)])

    plsc.subcore_barrier()

    for s_hbm, dst_hbm in zip(s_hbms, dst_hbms):
        es = dst_hbm.shape[0]
        epc = es // NC
        ept = epc // NS
        nch = ept // CH
        base0 = c * epc + s * ept

        def load_descs(k, b, s_hbm=s_hbm, dst_hbm=dst_hbm, base0=base0):
            base = base0 + k * CH
            return (pltpu.make_async_copy(s_hbm.at[pl.ds(base, CH)], sbuf[b],
                                          seml[b]),
                    pltpu.make_async_copy(dst_hbm.at[pl.ds(base, CH)],
                                          idxb[b], seml[b]))

        def step(k, b, load_descs=load_descs, nch=nch):
            for dsc in load_descs(k, b):
                dsc.wait()
            pltpu.sync_copy(sbuf[b], acc.at[idxb[b]], add=True)

            @pl.when(k + 2 < nch)
            def _():
                for dsc in load_descs(k + 2, b):
                    dsc.start()

        for b in range(2):
            for dsc in load_descs(b, b):
                dsc.start()

        def pair(p, _, step=step):
            step(2 * p, 0)
            step(2 * p + 1, 1)
            return 0

        lax.fori_loop(0, nch // 2, pair, 0)
        if nch % 2:
            step(nch - 1, 0)
    plsc.subcore_barrier()
    pltpu.sync_copy(acc.at[pl.ds(base_r, rpt)],
                    out_hbm.at[c, pl.ds(base_r, rpt)])

    @pl.when(s == NS - 1)
    def _():
        pltpu.sync_copy(acc.at[pl.ds(N - rem, rem)],
                        out_hbm.at[c, pl.ds(N - rem, rem)])


def _pq_body(h_ref, whi_ref, whj_ref, p_ref, q_ref):
    h = h_ref[...]
    p_ref[...] = jnp.dot(h, whi_ref[...], preferred_element_type=jnp.float32)
    q_ref[...] = jnp.dot(h, whj_ref[...], preferred_element_type=jnp.float32)


def _edge_body(g_ref, d2_ref, ea_ref, wa_ref, wr_ref, we2_ref, winf_ref,
               be1_ref, be2_ref, binf_ref, offs_ref, s_ref):
    nrow = d2_ref.shape[0]
    parts = []
    for j in range(nrow):
        d2j = d2_ref[j]
        dj = jnp.sqrt(jnp.maximum(d2j, 1e-12))
        rftj = jnp.exp(_COEFF * (dj - offs_ref[...]) ** 2)
        parts.append(lax.dot_general(
            rftj, wr_ref[...], (((0,), (0,)), ((), ())),
            preferred_element_type=jnp.float32))
    rfc = jnp.concatenate(parts, axis=0)
    pre = (jnp.dot(ea_ref[...], wa_ref[...], preferred_element_type=jnp.float32)
           + rfc + g_ref[...] + be1_ref[...])
    h1 = jnp.maximum(pre, 0.0)
    mij = jnp.maximum(
        jnp.dot(h1, we2_ref[...], preferred_element_type=jnp.float32)
        + be2_ref[...], 0.0)
    logit = (jnp.dot(mij, winf_ref[...], preferred_element_type=jnp.float32)
             + binf_ref[...])
    s_ref[...] = mij * jax.nn.sigmoid(logit)


def _node_body(m_ref, h_ref, wna_ref, wnb_ref, wn2_ref,
               bn1_ref, bn2_ref, o_ref):
    mi = m_ref[0] + m_ref[1]
    a = jnp.maximum(
        jnp.dot(mi, wna_ref[...], preferred_element_type=jnp.float32)
        + jnp.dot(h_ref[...], wnb_ref[...], preferred_element_type=jnp.float32)
        + bn1_ref[...], 0.0)
    o_ref[...] = (jnp.dot(a, wn2_ref[...], preferred_element_type=jnp.float32)
                  + bn2_ref[...])


def _full(shape):
    return pl.BlockSpec(shape, lambda i: (0,) * len(shape))


def kernel(h, x, edge_index, edge_attr, W_e1, b_e1, W_e2, b_e2, W_inf, b_inf,
           W_n1, b_n1, W_n2, b_n2):
    N, H = h.shape
    E = edge_index.shape[1]
    EF = edge_attr.shape[1]
    dst = edge_index[0]
    src = edge_index[1]

    W_a = W_e1[:EF]
    W_r = W_e1[EF:EF + NG]
    W_hi = W_e1[EF + NG:EF + NG + H]
    W_hj = W_e1[EF + NG + H:]

    f32 = jnp.float32
    BN = 1000

    P, Q = pl.pallas_call(
        _pq_body,
        grid=(N // BN,),
        in_specs=[pl.BlockSpec((BN, H), lambda i: (i, 0)),
                  _full((H, H)), _full((H, H))],
        out_specs=[pl.BlockSpec((BN, H), lambda i: (i, 0)),
                   pl.BlockSpec((BN, H), lambda i: (i, 0))],
        out_shape=[jax.ShapeDtypeStruct((N, H), f32),
                   jax.ShapeDtypeStruct((N, H), f32)],
    )(h, W_hi, W_hj)

    x0 = x[:, 0]
    x1 = x[:, 1]
    x2 = x[:, 2]

    mesh = plsc.VectorSubcoreMesh(core_axis_name="c", subcore_axis_name="s",
                                  num_cores=NC, num_subcores=NS)
    sc_params = pltpu.CompilerParams(needs_layout_passes=False)
    BE = 2560
    epw_pad = -(-(E // NW) // (2 * CHG)) * (2 * CHG)
    E_pad = NW * epw_pad
    zpad = jnp.zeros((E_pad - E,), jnp.int32)
    dstp = jnp.concatenate([dst, zpad])
    srcp = jnp.concatenate([src, zpad])

    G, d2 = pl.kernel(
        _gather_body,
        compiler_params=sc_params,
        out_type=(jax.ShapeDtypeStruct((E_pad, H), f32),
                  jax.ShapeDtypeStruct((E_pad,), f32)),
        mesh=mesh,
        scratch_types=[
            pltpu.VMEM((epw_pad,), jnp.int32),
            pltpu.VMEM((epw_pad,), jnp.int32),
            pltpu.VMEM((CHG, H), f32),
            pltpu.VMEM((CHG, H), f32),
            pltpu.VMEM((CHG, H), f32),
            pltpu.VMEM((CHG, H), f32),
            pltpu.VMEM((CHG,), f32),
            pltpu.VMEM((CHG,), f32),
            pltpu.VMEM((N,), f32),
            pltpu.VMEM((N,), f32),
            pltpu.VMEM((N,), f32),
            pltpu.SemaphoreType.DMA,
            pltpu.SemaphoreType.DMA,
            pltpu.SemaphoreType.DMA,
            pltpu.SemaphoreType.DMA,
        ],
    )(P, Q, x0, x1, x2, dstp, srcp)

    d2r = jnp.reshape(d2, (E_pad // 128, 1, 128))
    s = pl.pallas_call(
        _edge_body,
        grid=(E // BE,),
        in_specs=[pl.BlockSpec((BE, H), lambda i: (i, 0)),
                  pl.BlockSpec((BE // 128, 1, 128), lambda i: (i, 0, 0)),
                  pl.BlockSpec((BE, EF), lambda i: (i, 0)),
                  _full((EF, H)), _full((NG, H)), _full((H, H)),
                  _full((H, 128)), _full((1, H)), _full((1, H)),
                  _full((1, 128)), _full((NG, 128))],
        out_specs=pl.BlockSpec((BE, H), lambda i: (i, 0)),
        out_shape=jax.ShapeDtypeStruct((E, H), f32),
    )(G, d2r, edge_attr, W_a, W_r, W_e2,
      jnp.broadcast_to(W_inf, (H, 128)), b_e1.reshape(1, H),
      b_e2.reshape(1, H), jnp.broadcast_to(b_inf.reshape(1, 1), (1, 128)),
      jnp.broadcast_to(jnp.asarray(_OFFS).reshape(NG, 1), (NG, 128)))

    mi2 = pl.kernel(
        _scatter_body,
        compiler_params=sc_params,
        out_type=jax.ShapeDtypeStruct((NC, N, H), f32),
        mesh=mesh,
        scratch_types=[
            pltpu.VMEM((CH, H), f32),
            pltpu.VMEM((CH,), jnp.int32),
            pltpu.VMEM((CH, H), f32),
            pltpu.VMEM((CH,), jnp.int32),
            pltpu.VMEM((16, H), f32),
            pltpu.VMEM_SHARED((N, H), f32),
            pltpu.SemaphoreType.DMA,
            pltpu.SemaphoreType.DMA,
        ],
    )(s, dst)

    out = pl.pallas_call(
        _node_body,
        grid=(N // BN,),
        in_specs=[pl.BlockSpec((NC, BN, H), lambda i: (0, i, 0)),
                  pl.BlockSpec((BN, H), lambda i: (i, 0)),
                  _full((H, H)), _full((H, H)), _full((H, H)),
                  _full((1, H)), _full((1, H))],
        out_specs=pl.BlockSpec((BN, H), lambda i: (i, 0)),
        out_shape=jax.ShapeDtypeStruct((N, H), f32),
    )(mi2, h, W_n1[:H], W_n1[H:], W_n2,
      b_n1.reshape(1, H), b_n2.reshape(1, H))

    return (out, x)


# speedup vs baseline: 1.3356x; 1.1533x over previous
import functools

import jax
import jax.numpy as jnp
import numpy as np
from jax import lax
from jax.experimental import pallas as pl
from jax.experimental.pallas import tpu as pltpu
from jax.experimental.pallas import tpu_sc as plsc

NC = 2
NS = 16
NW = NC * NS
CH = 80

NSLAB = 1

NG = 16
_OFFS = np.linspace(0.0, 100.0, NG).astype(np.float32)
_COEFF = np.float32(-0.5 / (_OFFS[1] - _OFFS[0]) ** 2)


def _gather_body(p_hbm, q_hbm, x0_hbm, x1_hbm, x2_hbm, dst_hbm, src_hbm,
                 g_hbm, d2_hbm,
                 idxd, idxs, bufp0, bufq0, bufp1, bufq1, bufp2, bufq2,
                 d2v0, d2v1, d2v2, x0v, x1v, x2v,
                 semg0, semg1, semg2, semw0, semw1, semw2):
    E = dst_hbm.shape[0]
    epw = E // NW
    nch = epw // CH
    c = lax.axis_index("c")
    s = lax.axis_index("s")
    wid = s * NC + c
    base0 = wid * epw

    bufp = (bufp0, bufp1, bufp2)
    bufq = (bufq0, bufq1, bufq2)
    d2v = (d2v0, d2v1, d2v2)
    semg = (semg0, semg1, semg2)
    semw = (semw0, semw1, semw2)

    pltpu.sync_copy(x0_hbm, x0v)
    pltpu.sync_copy(x1_hbm, x1v)
    pltpu.sync_copy(x2_hbm, x2v)
    pltpu.sync_copy(dst_hbm.at[pl.ds(base0, epw)], idxd)
    pltpu.sync_copy(src_hbm.at[pl.ds(base0, epw)], idxs)

    def gather_descs(k, b):
        off = k * CH
        return (pltpu.make_async_copy(
                    p_hbm.at[idxd.at[pl.ds(off, CH)]], bufp[b], semg[b]),
                pltpu.make_async_copy(
                    q_hbm.at[idxs.at[pl.ds(off, CH)]], bufq[b], semg[b]))

    def wb_descs(k, b):
        base = base0 + k * CH
        return (pltpu.make_async_copy(bufp[b], g_hbm.at[pl.ds(base, CH)],
                                      semw[b]),
                pltpu.make_async_copy(d2v[b], d2_hbm.at[pl.ds(base, CH)],
                                      semw[b]))

    def process(k, b):
        for dsc in gather_descs(k, b):
            dsc.wait()
        bp, bq, dv = bufp[b], bufq[b], d2v[b]

        def row(r, _):
            for j in range(8):
                sl = pl.ds(j * 16, 16)
                bp[r, sl] = bp[r, sl] + bq[r, sl]
            return 0

        lax.fori_loop(0, CH, row, 0, unroll=2)

        def grp(g, _):
            sl = pl.ds(k * CH + g * 16, 16)
            dvx = idxd[sl]
            svx = idxs[sl]
            d0 = plsc.load_gather(x0v, [dvx]) - plsc.load_gather(x0v, [svx])
            d1 = plsc.load_gather(x1v, [dvx]) - plsc.load_gather(x1v, [svx])
            d2 = plsc.load_gather(x2v, [dvx]) - plsc.load_gather(x2v, [svx])
            dv[pl.ds(g * 16, 16)] = d0 * d0 + d1 * d1 + d2 * d2
            return 0

        lax.fori_loop(0, CH // 16, grp, 0, unroll=5)
        for dsc in wb_descs(k, b):
            dsc.start()

    ntrip = nch // 3
    assert nch == 3 * ntrip + 2
    for dsc in gather_descs(0, 0):
        dsc.start()
    for dsc in gather_descs(1, 1):
        dsc.start()

    def triple(t, _):
        k = 3 * t

        @pl.when(t > 0)
        def _():
            for dsc in wb_descs(k - 1, 2):
                dsc.wait()

        for dsc in gather_descs(k + 2, 2):
            dsc.start()
        process(k, 0)

        for dsc in wb_descs(k, 0):
            dsc.wait()
        for dsc in gather_descs(k + 3, 0):
            dsc.start()
        process(k + 1, 1)

        for dsc in wb_descs(k + 1, 1):
            dsc.wait()
        for dsc in gather_descs(k + 4, 1):
            dsc.start()
        process(k + 2, 2)
        return 0

    lax.fori_loop(0, ntrip, triple, 0)
    process(nch - 2, 0)
    process(nch - 1, 1)
    for dsc in wb_descs(nch - 3, 2):
        dsc.wait()
    for dsc in wb_descs(nch - 2, 0):
        dsc.wait()
    for dsc in wb_descs(nch - 1, 1):
        dsc.wait()


def _scatter_body(*refs):
    s_hbms = refs[:NSLAB]
    dst_hbms = refs[NSLAB:2 * NSLAB]
    out_hbm = refs[2 * NSLAB]
    sbuf0, idxb0, sbuf1, idxb1, zb, acc, seml0, seml1 = refs[2 * NSLAB + 1:]
    sbuf = (sbuf0, sbuf1)
    idxb = (idxb0, idxb1)
    seml = (seml0, seml1)
    N = acc.shape[0]
    rpt = 8 * (N // (8 * NS))
    rem = N - rpt * NS
    c = lax.axis_index("c")
    s = lax.axis_index("s")
    base_r = s * rpt

    def zrow(r, _):
        for k in range(8):
            zb[r, pl.ds(k * 16, 16)] = jnp.zeros((16,), jnp.float32)
        return 0

    lax.fori_loop(0, zb.shape[0], zrow, 0)

    def zcopy(j, _):
        pltpu.sync_copy(zb, acc.at[pl.ds(base_r + j * zb.shape[0],
                                         zb.shape[0])])
        return 0

    lax.fori_loop(0, rpt // zb.shape[0], zcopy, 0)

    @pl.when(s == NS - 1)
    def _():
        pltpu.sync_copy(zb, acc.at[pl.ds(N - rem, rem)])

    plsc.subcore_barrier()

    for s_hbm, dst_hbm in zip(s_hbms, dst_hbms):
        es = dst_hbm.shape[0]
        epc = es // NC
        ept = epc // NS
        nch = ept // CH
        base0 = c * epc + s * ept

        def load_descs(k, b, s_hbm=s_hbm, dst_hbm=dst_hbm, base0=base0):
            base = base0 + k * CH
            return (pltpu.make_async_copy(s_hbm.at[pl.ds(base, CH)], sbuf[b],
                                          seml[b]),
                    pltpu.make_async_copy(dst_hbm.at[pl.ds(base, CH)],
                                          idxb[b], seml[b]))

        def step(k, b, load_descs=load_descs, nch=nch):
            for dsc in load_descs(k, b):
                dsc.wait()
            pltpu.sync_copy(sbuf[b], acc.at[idxb[b]], add=True)

            @pl.when(k + 2 < nch)
            def _():
                for dsc in load_descs(k + 2, b):
                    dsc.start()

        for b in range(2):
            for dsc in load_descs(b, b):
                dsc.start()

        def pair(p, _, step=step):
            step(2 * p, 0)
            step(2 * p + 1, 1)
            return 0

        lax.fori_loop(0, nch // 2, pair, 0)
        if nch % 2:
            step(nch - 1, 0)
    plsc.subcore_barrier()
    pltpu.sync_copy(acc.at[pl.ds(base_r, rpt)],
                    out_hbm.at[c, pl.ds(base_r, rpt)])

    @pl.when(s == NS - 1)
    def _():
        pltpu.sync_copy(acc.at[pl.ds(N - rem, rem)],
                        out_hbm.at[c, pl.ds(N - rem, rem)])


def _pq_body(h_ref, whi_ref, whj_ref, p_ref, q_ref):
    h = h_ref[...]
    p_ref[...] = jnp.dot(h, whi_ref[...], preferred_element_type=jnp.float32)
    q_ref[...] = jnp.dot(h, whj_ref[...], preferred_element_type=jnp.float32)


def _edge_body(g_ref, d2_ref, ea_ref, wa_ref, wr_ref, we2_ref, winf_ref,
               be1_ref, be2_ref, binf_ref, offs_ref, s_ref):
    nrow = d2_ref.shape[0]
    parts = []
    for j in range(nrow):
        d2j = d2_ref[j]
        dj = jnp.sqrt(jnp.maximum(d2j, 1e-12))
        rftj = jnp.exp(_COEFF * (dj - offs_ref[...]) ** 2)
        parts.append(lax.dot_general(
            rftj, wr_ref[...], (((0,), (0,)), ((), ())),
            preferred_element_type=jnp.float32))
    rfc = jnp.concatenate(parts, axis=0)
    pre = (jnp.dot(ea_ref[...], wa_ref[...], preferred_element_type=jnp.float32)
           + rfc + g_ref[...] + be1_ref[...])
    h1 = jnp.maximum(pre, 0.0)
    mij = jnp.maximum(
        jnp.dot(h1, we2_ref[...], preferred_element_type=jnp.float32)
        + be2_ref[...], 0.0)
    logit = (jnp.dot(mij, winf_ref[...], preferred_element_type=jnp.float32)
             + binf_ref[...])
    s_ref[...] = mij * jax.nn.sigmoid(logit)


def _node_body(m_ref, h_ref, wna_ref, wnb_ref, wn2_ref,
               bn1_ref, bn2_ref, o_ref):
    mi = m_ref[0] + m_ref[1]
    a = jnp.maximum(
        jnp.dot(mi, wna_ref[...], preferred_element_type=jnp.float32)
        + jnp.dot(h_ref[...], wnb_ref[...], preferred_element_type=jnp.float32)
        + bn1_ref[...], 0.0)
    o_ref[...] = (jnp.dot(a, wn2_ref[...], preferred_element_type=jnp.float32)
                  + bn2_ref[...])


def _full(shape):
    return pl.BlockSpec(shape, lambda i: (0,) * len(shape))


def kernel(h, x, edge_index, edge_attr, W_e1, b_e1, W_e2, b_e2, W_inf, b_inf,
           W_n1, b_n1, W_n2, b_n2):
    N, H = h.shape
    E = edge_index.shape[1]
    EF = edge_attr.shape[1]
    dst = edge_index[0]
    src = edge_index[1]

    W_a = W_e1[:EF]
    W_r = W_e1[EF:EF + NG]
    W_hi = W_e1[EF + NG:EF + NG + H]
    W_hj = W_e1[EF + NG + H:]

    f32 = jnp.float32
    BN = 1000

    P, Q = pl.pallas_call(
        _pq_body,
        grid=(N // BN,),
        in_specs=[pl.BlockSpec((BN, H), lambda i: (i, 0)),
                  _full((H, H)), _full((H, H))],
        out_specs=[pl.BlockSpec((BN, H), lambda i: (i, 0)),
                   pl.BlockSpec((BN, H), lambda i: (i, 0))],
        out_shape=[jax.ShapeDtypeStruct((N, H), f32),
                   jax.ShapeDtypeStruct((N, H), f32)],
    )(h, W_hi, W_hj)

    x0 = x[:, 0]
    x1 = x[:, 1]
    x2 = x[:, 2]

    mesh = plsc.VectorSubcoreMesh(core_axis_name="c", subcore_axis_name="s",
                                  num_cores=NC, num_subcores=NS)
    sc_params = pltpu.CompilerParams(needs_layout_passes=False)
    ES = E // NSLAB
    BE = 2560

    gather_fn = pl.kernel(
        _gather_body,
        compiler_params=sc_params,
        out_type=(jax.ShapeDtypeStruct((ES, H), f32),
                  jax.ShapeDtypeStruct((ES,), f32)),
        mesh=mesh,
        scratch_types=[
            pltpu.VMEM((ES // NW,), jnp.int32),
            pltpu.VMEM((ES // NW,), jnp.int32),
            pltpu.VMEM((CH, H), f32),
            pltpu.VMEM((CH, H), f32),
            pltpu.VMEM((CH, H), f32),
            pltpu.VMEM((CH, H), f32),
            pltpu.VMEM((CH, H), f32),
            pltpu.VMEM((CH, H), f32),
            pltpu.VMEM((CH,), f32),
            pltpu.VMEM((CH,), f32),
            pltpu.VMEM((CH,), f32),
            pltpu.VMEM((N,), f32),
            pltpu.VMEM((N,), f32),
            pltpu.VMEM((N,), f32),
            pltpu.SemaphoreType.DMA,
            pltpu.SemaphoreType.DMA,
            pltpu.SemaphoreType.DMA,
            pltpu.SemaphoreType.DMA,
            pltpu.SemaphoreType.DMA,
            pltpu.SemaphoreType.DMA,
        ],
    )

    edge_fn = pl.pallas_call(
        _edge_body,
        grid=(ES // BE,),
        in_specs=[pl.BlockSpec((BE, H), lambda i: (i, 0)),
                  pl.BlockSpec((BE // 128, 1, 128), lambda i: (i, 0, 0)),
                  pl.BlockSpec((BE, EF), lambda i: (i, 0)),
                  _full((EF, H)), _full((NG, H)), _full((H, H)),
                  _full((H, 128)), _full((1, H)), _full((1, H)),
                  _full((1, 128)), _full((NG, 128))],
        out_specs=pl.BlockSpec((BE, H), lambda i: (i, 0)),
        out_shape=jax.ShapeDtypeStruct((ES, H), f32),
    )

    winf_bc = jnp.broadcast_to(W_inf, (H, 128))
    binf_bc = jnp.broadcast_to(b_inf.reshape(1, 1), (1, 128))
    offs_bc = jnp.broadcast_to(jnp.asarray(_OFFS).reshape(NG, 1), (NG, 128))

    dsts = [dst[k * ES:(k + 1) * ES] for k in range(NSLAB)]
    srcs = [src[k * ES:(k + 1) * ES] for k in range(NSLAB)]
    ss = []
    for k in range(NSLAB):
        Gk, d2k = gather_fn(P, Q, x0, x1, x2, dsts[k], srcs[k])
        d2k = jnp.reshape(d2k, (ES // 128, 1, 128))
        ek = edge_fn(Gk, d2k, edge_attr[k * ES:(k + 1) * ES], W_a, W_r,
                     W_e2, winf_bc, b_e1.reshape(1, H), b_e2.reshape(1, H),
                     binf_bc, offs_bc)
        ss.append(ek)

    mi2 = pl.kernel(
        _scatter_body,
        compiler_params=sc_params,
        out_type=jax.ShapeDtypeStruct((NC, N, H), f32),
        mesh=mesh,
        scratch_types=[
            pltpu.VMEM((CH, H), f32),
            pltpu.VMEM((CH,), jnp.int32),
            pltpu.VMEM((CH, H), f32),
            pltpu.VMEM((CH,), jnp.int32),
            pltpu.VMEM((16, H), f32),
            pltpu.VMEM_SHARED((N, H), f32),
            pltpu.SemaphoreType.DMA,
            pltpu.SemaphoreType.DMA,
        ],
    )(*ss, *dsts)

    out = pl.pallas_call(
        _node_body,
        grid=(N // BN,),
        in_specs=[pl.BlockSpec((NC, BN, H), lambda i: (0, i, 0)),
                  pl.BlockSpec((BN, H), lambda i: (i, 0)),
                  _full((H, H)), _full((H, H)), _full((H, H)),
                  _full((1, H)), _full((1, H))],
        out_specs=pl.BlockSpec((BN, H), lambda i: (i, 0)),
        out_shape=jax.ShapeDtypeStruct((N, H), f32),
    )(mi2, h, W_n1[:H], W_n1[H:], W_n2,
      b_n1.reshape(1, H), b_n2.reshape(1, H))

    return (out, x)
